# baseline (speedup 1.0000x reference)
import functools

import jax
import jax.numpy as jnp
from jax import lax
from jax.experimental import pallas as pl
from jax.experimental.pallas import tpu as pltpu
from jax.experimental.pallas import tpu_sc as plsc

NC, NS = 2, 16
NW = NC * NS
CHUNK = 128


def _make_gather(n_chunks, chunks_per_w, D):
    mesh = plsc.VectorSubcoreMesh(core_axis_name="c", subcore_axis_name="s")

    @functools.partial(
        pl.kernel,
        out_type=jax.ShapeDtypeStruct((n_chunks * CHUNK, D), jnp.float32),
        mesh=mesh,
        scratch_types=[
            pltpu.VMEM((chunks_per_w, CHUNK), jnp.int32),
            pltpu.VMEM((CHUNK, D), jnp.float32),
            pltpu.SemaphoreType.DMA,
        ],
        compiler_params=pltpu.CompilerParams(use_tc_tiling_on_sc=False),
    )
    def run(idx_hbm, table_hbm, out_hbm, idx_v, rows_v, gsem):
        wid = lax.axis_index("s") * NC + lax.axis_index("c")
        row0 = wid * chunks_per_w
        pltpu.sync_copy(idx_hbm.at[pl.ds(row0, chunks_per_w)], idx_v)

        def body(c, _):
            pltpu.async_copy(table_hbm.at[idx_v.at[c]], rows_v, gsem).wait()
            pltpu.sync_copy(
                rows_v, out_hbm.at[pl.ds((row0 + c) * CHUNK, CHUNK)])
            return ()

        lax.fori_loop(0, chunks_per_w, body, ())

    return run


def kernel(cat_idx, table):
    B, F = cat_idx.shape
    V, D = table.shape
    total = B * F
    n_chunks = total // CHUNK
    chunks_per_w = n_chunks // NW
    idx2d = cat_idx.reshape(n_chunks, CHUNK).astype(jnp.int32)
    out = _make_gather(n_chunks, chunks_per_w, D)(idx2d, table)
    return out.reshape(B, F, D)


# speedup vs baseline: 1.0768x; 1.0768x over previous
import functools

import jax
import jax.numpy as jnp
from jax import lax
from jax.experimental import pallas as pl
from jax.experimental.pallas import tpu as pltpu
from jax.experimental.pallas import tpu_sc as plsc

NC, NS = 2, 16
NW = NC * NS
CHUNK = 128


NBUF = 8


def _make_gather(n_chunks, chunks_per_w, D):
    mesh = plsc.VectorSubcoreMesh(core_axis_name="c", subcore_axis_name="s")

    @functools.partial(
        pl.kernel,
        out_type=jax.ShapeDtypeStruct((n_chunks * CHUNK, D), jnp.float32),
        mesh=mesh,
        scratch_types=[
            pltpu.VMEM((chunks_per_w, CHUNK), jnp.int32),
            pltpu.VMEM((NBUF, CHUNK, D), jnp.float32),
            pltpu.SemaphoreType.DMA,
            pltpu.SemaphoreType.DMA,
        ],
        compiler_params=pltpu.CompilerParams(use_tc_tiling_on_sc=False),
    )
    def run(idx_hbm, table_hbm, out_hbm, idx_v, rows_v, gsem, osem):
        wid = lax.axis_index("s") * NC + lax.axis_index("c")
        row0 = wid * chunks_per_w
        pltpu.sync_copy(idx_hbm.at[pl.ds(row0, chunks_per_w)], idx_v)

        def fire_gather(c):
            pltpu.async_copy(
                table_hbm.at[idx_v.at[c]], rows_v.at[c % NBUF], gsem)

        def fire_wb(c):
            pltpu.async_copy(
                rows_v.at[c % NBUF],
                out_hbm.at[pl.ds((row0 + c) * CHUNK, CHUNK)], osem)

        def drain(sem):
            pltpu.make_async_copy(
                out_hbm.at[pl.ds(0, CHUNK)], rows_v.at[0], sem).wait()

        for c in range(NBUF - 2):
            fire_gather(c)

        def body(c, _):
            drain(gsem)
            fire_wb(c)
            @pl.when(c >= 2)
            def _():
                drain(osem)
            @pl.when(c + NBUF - 2 < chunks_per_w)
            def _():
                fire_gather(c + NBUF - 2)
            return ()

        lax.fori_loop(0, chunks_per_w, body, ())
        drain(osem)
        drain(osem)

    return run


def kernel(cat_idx, table):
    B, F = cat_idx.shape
    V, D = table.shape
    total = B * F
    n_chunks = total // CHUNK
    chunks_per_w = n_chunks // NW
    idx2d = cat_idx.reshape(n_chunks, CHUNK).astype(jnp.int32)
    out = _make_gather(n_chunks, chunks_per_w, D)(idx2d, table)
    return out.reshape(B, F, D)
